# baseline (speedup 1.0000x reference)
import functools

import jax
import jax.numpy as jnp
from jax.experimental import pallas as pl

K = 32
LANES = 128
DEPTH = 8
SENTINEL = -1.0


def _enc_body(x_ref, w_ref, b_ref, o_ref):
    o_ref[...] = (
        jnp.dot(x_ref[...], w_ref[...], preferred_element_type=jnp.float32)
        + b_ref[...]
    )


def _encode(x, W_enc, b_enc2d, tile_m, tile_n):
    n_tok, d_model = x.shape
    d_sae = W_enc.shape[1]
    grid = (n_tok // tile_m, d_sae // tile_n)
    return pl.pallas_call(
        _enc_body,
        grid=grid,
        in_specs=[
            pl.BlockSpec((tile_m, d_model), lambda t, n: (t, 0)),
            pl.BlockSpec((d_model, tile_n), lambda t, n: (0, n)),
            pl.BlockSpec((1, tile_n), lambda t, n: (0, n)),
        ],
        out_specs=pl.BlockSpec((tile_m, tile_n), lambda t, n: (t, n)),
        out_shape=jax.ShapeDtypeStruct((n_tok, d_sae), jnp.float32),
    )(x, W_enc, b_enc2d)


def _bitonic_clean(v):
    n = len(v)
    if n == 1:
        return v
    h = n // 2
    hi = [jnp.maximum(v[i], v[i + h]) for i in range(h)]
    lo = [jnp.minimum(v[i], v[i + h]) for i in range(h)]
    return _bitonic_clean(hi) + _bitonic_clean(lo)


def _merge_sorted(A, B):
    n = len(A)
    if 2 * n <= DEPTH:
        return _bitonic_clean(A + B[::-1])
    h = [jnp.maximum(A[i], B[n - 1 - i]) for i in range(n)]
    return _bitonic_clean(h)


def _tournament(cols):
    if len(cols) == 1:
        return cols
    h = len(cols) // 2
    return _merge_sorted(_tournament(cols[:h]), _tournament(cols[h:]))


def _topk_mask_decode_body(z_ref, wdec_ref, bdec_ref, zs_ref, xr_ref, *, d_sae):
    z = jnp.maximum(z_ref[...], 0.0)
    ncols = d_sae // LANES

    s = _tournament([z[:, j * LANES:(j + 1) * LANES] for j in range(ncols)])
    depth = len(s)

    sT = [jnp.transpose(x) for x in s]
    rows = sT[0].shape[1]
    cum = jnp.zeros((1, rows), jnp.float32)
    t32 = jnp.zeros((1, rows), jnp.float32)
    for _ in range(K):
        m = jnp.max(sT[0], axis=0, keepdims=True)
        eq = sT[0] == m
        cnt = jnp.sum(eq.astype(jnp.float32), axis=0, keepdims=True)
        t32 = jnp.where(cum < K, m, t32)
        cum = cum + cnt
        for d in range(depth - 1):
            sT[d] = jnp.where(eq, sT[d + 1], sT[d])
        sT[depth - 1] = jnp.where(eq, SENTINEL, sT[depth - 1])

    t32r = jnp.transpose(t32)
    zs = jnp.where((z >= t32r) & (z > 0.0), z, 0.0)
    zs_ref[...] = zs
    xr_ref[...] = (
        jnp.dot(
            zs.astype(jnp.bfloat16), wdec_ref[...],
            preferred_element_type=jnp.float32,
        )
        + bdec_ref[...]
    )


def _topk_mask_decode(z_pre, W_dec_bf16, b_dec2d, tile_rows):
    n_tok, d_sae = z_pre.shape
    d_model = W_dec_bf16.shape[1]
    grid = (n_tok // tile_rows,)
    return pl.pallas_call(
        functools.partial(_topk_mask_decode_body, d_sae=d_sae),
        grid=grid,
        in_specs=[
            pl.BlockSpec((tile_rows, d_sae), lambda i: (i, 0)),
            pl.BlockSpec((d_sae, d_model), lambda i: (0, 0)),
            pl.BlockSpec((1, d_model), lambda i: (0, 0)),
        ],
        out_specs=[
            pl.BlockSpec((tile_rows, d_sae), lambda i: (i, 0)),
            pl.BlockSpec((tile_rows, d_model), lambda i: (i, 0)),
        ],
        out_shape=[
            jax.ShapeDtypeStruct((n_tok, d_sae), jnp.float32),
            jax.ShapeDtypeStruct((n_tok, d_model), jnp.float32),
        ],
    )(z_pre, W_dec_bf16, b_dec2d)


def kernel(x, W_enc, b_enc, W_dec, b_dec):
    n_tok, d_model = x.shape
    d_sae = W_enc.shape[1]
    tile_n = min(1024, d_sae)
    tile_rows = min(64, n_tok)
    tile_m = min(2048, n_tok)

    z_pre = _encode(x, W_enc, b_enc.reshape(1, d_sae), tile_m, tile_n)
    z_sparse, x_recon = _topk_mask_decode(
        z_pre, W_dec.astype(jnp.bfloat16), b_dec.reshape(1, d_model), tile_rows
    )
    return (x_recon, z_sparse, z_pre)


# speedup vs baseline: 1.5093x; 1.5093x over previous
import functools

import jax
import jax.numpy as jnp
from jax.experimental import pallas as pl

K = 32
LANES = 128
DEPTH = 8
SENTINEL = -1.0


def _enc_body(x_ref, w_ref, b_ref, o_ref):
    o_ref[...] = (
        jnp.dot(x_ref[...], w_ref[...], preferred_element_type=jnp.float32)
        + b_ref[...]
    )


def _encode(x, W_enc, b_enc2d, tile_m, tile_n):
    n_tok, d_model = x.shape
    d_sae = W_enc.shape[1]
    grid = (n_tok // tile_m, d_sae // tile_n)
    return pl.pallas_call(
        _enc_body,
        grid=grid,
        in_specs=[
            pl.BlockSpec((tile_m, d_model), lambda t, n: (t, 0)),
            pl.BlockSpec((d_model, tile_n), lambda t, n: (0, n)),
            pl.BlockSpec((1, tile_n), lambda t, n: (0, n)),
        ],
        out_specs=pl.BlockSpec((tile_m, tile_n), lambda t, n: (t, n)),
        out_shape=jax.ShapeDtypeStruct((n_tok, d_sae), jnp.float32),
    )(x, W_enc, b_enc2d)


def _bitonic_clean(v):
    n = len(v)
    if n == 1:
        return v
    h = n // 2
    hi = [jnp.maximum(v[i], v[i + h]) for i in range(h)]
    lo = [jnp.minimum(v[i], v[i + h]) for i in range(h)]
    return _bitonic_clean(hi) + _bitonic_clean(lo)


def _merge_sorted(A, B):
    n = len(A)
    if 2 * n <= DEPTH:
        return _bitonic_clean(A + B[::-1])
    h = [jnp.maximum(A[i], B[n - 1 - i]) for i in range(n)]
    return _bitonic_clean(h)


def _tournament(cols):
    if len(cols) == 1:
        return cols
    h = len(cols) // 2
    return _merge_sorted(_tournament(cols[:h]), _tournament(cols[h:]))


def _topk_body(z_ref, zs_ref, *, d_sae):
    z = jnp.maximum(z_ref[...], 0.0)
    ncols = d_sae // LANES

    s = _tournament([z[:, j * LANES:(j + 1) * LANES] for j in range(ncols)])
    depth = len(s)

    sT = [jnp.transpose(x) for x in s]
    rows = sT[0].shape[1]
    cum = jnp.zeros((1, rows), jnp.float32)
    t32 = jnp.zeros((1, rows), jnp.float32)
    for _ in range(K):
        m = jnp.max(sT[0], axis=0, keepdims=True)
        eq = sT[0] == m
        cnt = jnp.sum(eq.astype(jnp.float32), axis=0, keepdims=True)
        t32 = jnp.where(cum < K, m, t32)
        cum = cum + cnt
        for d in range(depth - 1):
            sT[d] = jnp.where(eq, sT[d + 1], sT[d])
        sT[depth - 1] = jnp.where(eq, SENTINEL, sT[depth - 1])

    t32r = jnp.transpose(t32)
    zs_ref[...] = jnp.where((z >= t32r) & (z > 0.0), z, 0.0)


def _topk_mask(z_pre, tile_rows):
    n_tok, d_sae = z_pre.shape
    grid = (n_tok // tile_rows,)
    return pl.pallas_call(
        functools.partial(_topk_body, d_sae=d_sae),
        grid=grid,
        in_specs=[pl.BlockSpec((tile_rows, d_sae), lambda i: (i, 0))],
        out_specs=pl.BlockSpec((tile_rows, d_sae), lambda i: (i, 0)),
        out_shape=jax.ShapeDtypeStruct((n_tok, d_sae), jnp.float32),
    )(z_pre)


def _dec_body(zs_ref, w_ref, b_ref, o_ref):
    @pl.when(pl.program_id(0) == 0)
    def _init():
        o_ref[...] = jnp.broadcast_to(b_ref[...], o_ref.shape)

    o_ref[...] += jnp.dot(
        zs_ref[...], w_ref[...], preferred_element_type=jnp.float32
    )


def _decode(z_sparse, W_dec, b_dec2d, tile_k):
    n_tok, d_sae = z_sparse.shape
    d_model = W_dec.shape[1]
    grid = (d_sae // tile_k,)
    return pl.pallas_call(
        _dec_body,
        grid=grid,
        in_specs=[
            pl.BlockSpec((n_tok, tile_k), lambda k: (0, k)),
            pl.BlockSpec((tile_k, d_model), lambda k: (k, 0)),
            pl.BlockSpec((1, d_model), lambda k: (0, 0)),
        ],
        out_specs=pl.BlockSpec((n_tok, d_model), lambda k: (0, 0)),
        out_shape=jax.ShapeDtypeStruct((n_tok, d_model), jnp.float32),
    )(z_sparse, W_dec, b_dec2d)


def kernel(x, W_enc, b_enc, W_dec, b_dec):
    n_tok, d_model = x.shape
    d_sae = W_enc.shape[1]
    tile_n = min(1024, d_sae)
    tile_rows = min(128, n_tok)
    tile_k = min(512, d_sae)
    tile_m = min(2048, n_tok)

    z_pre = _encode(x, W_enc, b_enc.reshape(1, d_sae), tile_m, tile_n)
    z_sparse = _topk_mask(z_pre, tile_rows)
    x_recon = _decode(z_sparse, W_dec, b_dec.reshape(1, d_model), tile_k)
    return (x_recon, z_sparse, z_pre)


# speedup vs baseline: 1.5957x; 1.0573x over previous
import functools

import jax
import jax.numpy as jnp
from jax.experimental import pallas as pl

K = 32
LANES = 128
DEPTH = 8
SENTINEL = -1.0


def _enc_body(x_ref, w_ref, b_ref, o_ref):
    o_ref[...] = (
        jnp.dot(x_ref[...], w_ref[...], preferred_element_type=jnp.float32)
        + b_ref[...]
    )


def _encode(x, W_enc, b_enc2d, tile_n):
    n_tok, d_model = x.shape
    d_sae = W_enc.shape[1]
    grid = (d_sae // tile_n,)
    return pl.pallas_call(
        _enc_body,
        grid=grid,
        in_specs=[
            pl.BlockSpec((n_tok, d_model), lambda n: (0, 0)),
            pl.BlockSpec((d_model, tile_n), lambda n: (0, n)),
            pl.BlockSpec((1, tile_n), lambda n: (0, n)),
        ],
        out_specs=pl.BlockSpec((n_tok, tile_n), lambda n: (0, n)),
        out_shape=jax.ShapeDtypeStruct((n_tok, d_sae), jnp.float32),
    )(x, W_enc, b_enc2d)


def _bitonic_clean(v):
    n = len(v)
    if n == 1:
        return v
    h = n // 2
    hi = [jnp.maximum(v[i], v[i + h]) for i in range(h)]
    lo = [jnp.minimum(v[i], v[i + h]) for i in range(h)]
    return _bitonic_clean(hi) + _bitonic_clean(lo)


def _merge_sorted(A, B):
    n = len(A)
    if 2 * n <= DEPTH:
        return _bitonic_clean(A + B[::-1])
    h = [jnp.maximum(A[i], B[n - 1 - i]) for i in range(n)]
    return _bitonic_clean(h)


def _tournament(cols):
    if len(cols) == 1:
        return cols
    h = len(cols) // 2
    return _merge_sorted(_tournament(cols[:h]), _tournament(cols[h:]))


def _topk_body(z_ref, zs_ref, *, d_sae):
    z = z_ref[...]
    ncols = d_sae // LANES

    s = _tournament([z[:, j * LANES:(j + 1) * LANES] for j in range(ncols)])
    depth = len(s)

    sT = [jnp.transpose(x) for x in s]
    rows = sT[0].shape[1]
    cum = jnp.zeros((1, rows), jnp.float32)
    t32 = jnp.zeros((1, rows), jnp.float32)
    for _ in range(K):
        m = jnp.max(sT[0], axis=0, keepdims=True)
        eq = sT[0] == m
        cnt = jnp.sum(eq.astype(jnp.float32), axis=0, keepdims=True)
        t32 = jnp.where(cum < K, m, t32)
        cum = cum + cnt
        for d in range(depth - 1):
            sT[d] = jnp.where(eq, sT[d + 1], sT[d])
        sT[depth - 1] = jnp.where(eq, SENTINEL, sT[depth - 1])

    t32c = jnp.maximum(jnp.transpose(t32), jnp.finfo(jnp.float32).tiny)
    zs_ref[...] = jnp.where(z >= t32c, z, 0.0)


def _topk_mask(z_pre, tile_rows):
    n_tok, d_sae = z_pre.shape
    grid = (n_tok // tile_rows,)
    return pl.pallas_call(
        functools.partial(_topk_body, d_sae=d_sae),
        grid=grid,
        in_specs=[pl.BlockSpec((tile_rows, d_sae), lambda i: (i, 0))],
        out_specs=pl.BlockSpec((tile_rows, d_sae), lambda i: (i, 0)),
        out_shape=jax.ShapeDtypeStruct((n_tok, d_sae), jnp.float32),
    )(z_pre)


def _dec_body(zs_ref, w_ref, b_ref, o_ref):
    @pl.when(pl.program_id(0) == 0)
    def _init():
        o_ref[...] = jnp.broadcast_to(b_ref[...], o_ref.shape)

    o_ref[...] += jnp.dot(
        zs_ref[...], w_ref[...], preferred_element_type=jnp.float32
    )


def _decode(z_sparse, W_dec, b_dec2d, tile_k):
    n_tok, d_sae = z_sparse.shape
    d_model = W_dec.shape[1]
    grid = (d_sae // tile_k,)
    return pl.pallas_call(
        _dec_body,
        grid=grid,
        in_specs=[
            pl.BlockSpec((n_tok, tile_k), lambda k: (0, k)),
            pl.BlockSpec((tile_k, d_model), lambda k: (k, 0)),
            pl.BlockSpec((1, d_model), lambda k: (0, 0)),
        ],
        out_specs=pl.BlockSpec((n_tok, d_model), lambda k: (0, 0)),
        out_shape=jax.ShapeDtypeStruct((n_tok, d_model), jnp.float32),
    )(z_sparse, W_dec, b_dec2d)


def kernel(x, W_enc, b_enc, W_dec, b_dec):
    n_tok, d_model = x.shape
    d_sae = W_enc.shape[1]
    tile_n = min(512, d_sae)
    tile_rows = min(128, n_tok)
    tile_k = min(512, d_sae)

    z_pre = _encode(x, W_enc, b_enc.reshape(1, d_sae), tile_n)
    z_sparse = _topk_mask(z_pre, tile_rows)
    x_recon = _decode(z_sparse, W_dec, b_dec.reshape(1, d_model), tile_k)
    return (x_recon, z_sparse, z_pre)
